# baseline (speedup 1.0000x reference)
import functools

import jax
import jax.numpy as jnp
from jax import lax
from jax.experimental import pallas as pl
from jax.experimental.pallas import tpu as pltpu
from jax.experimental.pallas import tpu_sc as plsc

D = 128
N = 512
TOTAL = N * N
NW = 32
PER_W = TOTAL // NW
CHUNK = 64
NCHUNK = PER_W // CHUNK
NBUF = 8
L = 16
TSTRIDE = 72


def _make_kernel():
    mesh = plsc.VectorSubcoreMesh(core_axis_name="c", subcore_axis_name="s")

    scratch = [
        pltpu.VMEM((PER_W,), jnp.int32),
        pltpu.VMEM((PER_W,), jnp.int32),
        pltpu.VMEM_SHARED((16 * TSTRIDE, D), jnp.float32),
    ]
    scratch += [pltpu.VMEM((CHUNK, D), jnp.float32) for _ in range(NBUF)]
    scratch += [pltpu.SemaphoreType.DMA for _ in range(2 * NBUF)]

    @functools.partial(
        pl.kernel,
        mesh=mesh,
        out_type=jax.ShapeDtypeStruct((TOTAL, D), jnp.float32),
        scratch_types=scratch,
    )
    def k(templ_hbm, mask_hbm, table_hbm, out_hbm, templ_v, idx_v, table_v,
          *bufs_and_sems):
        rows = bufs_and_sems[:NBUF]
        gsem = bufs_and_sems[NBUF:2 * NBUF]
        ssem = bufs_and_sems[2 * NBUF:]
        sid = lax.axis_index("s")
        wid = sid * 2 + lax.axis_index("c")
        base = wid * PER_W

        pltpu.sync_copy(table_hbm, table_v.at[pl.ds(sid * TSTRIDE, 65)])
        pltpu.sync_copy(templ_hbm.at[pl.ds(base, PER_W)], templ_v)
        pltpu.sync_copy(mask_hbm.at[pl.ds(base, PER_W)], idx_v)
        plsc.subcore_barrier()

        off = sid * TSTRIDE

        def compute_idx(i, carry):
            t = templ_v[pl.ds(i * L, L)]
            m = idx_v[pl.ds(i * L, L)]
            idx_v[pl.ds(i * L, L)] = m * (t + 1) + off
            return carry
        lax.fori_loop(0, PER_W // L, compute_idx, 0)

        @pl.loop(0, NCHUNK, step=NBUF)
        def _(c0):
            handles = []
            for b in range(NBUF):
                @pl.when(c0 > 0)
                def _():
                    pltpu.make_async_copy(
                        rows[b], out_hbm.at[pl.ds(0, CHUNK)], ssem[b]
                    ).wait()
                idx_c = idx_v.at[pl.ds((c0 + b) * CHUNK, CHUNK)]
                handles.append(
                    pltpu.async_copy(table_v.at[idx_c], rows[b], gsem[b]))
            for b in range(NBUF):
                handles[b].wait()
                pltpu.async_copy(
                    rows[b],
                    out_hbm.at[pl.ds(base + (c0 + b) * CHUNK, CHUNK)],
                    ssem[b],
                )
        for b in range(NBUF):
            pltpu.make_async_copy(
                rows[b], out_hbm.at[pl.ds(0, CHUNK)], ssem[b]
            ).wait()

    return k


_embed = _make_kernel()


def kernel(conditional_templ, conditional_templ_mask, table):
    out = _embed(conditional_templ.reshape(TOTAL),
                 conditional_templ_mask.reshape(TOTAL),
                 table)
    return out.reshape(N, N, D)
